# baseline (speedup 1.0000x reference)
import jax
import jax.numpy as jnp
from jax.experimental import pallas as pl
from jax.experimental.pallas import tpu as pltpu

EPS = 1e-12


def _netvlad_body(x_ref, w_ref, c_ref, out_ref):
    xb = x_ref[0]
    n2 = jnp.sum(xb * xb, axis=0, keepdims=True)
    xb = xb / jnp.maximum(jnp.sqrt(n2), EPS)
    logits = jnp.dot(w_ref[...], xb, preferred_element_type=jnp.float32)
    m = jnp.max(logits, axis=0, keepdims=True)
    e = jnp.exp(logits - m)
    a = e / jnp.sum(e, axis=0, keepdims=True)
    agg = jax.lax.dot_general(
        a, xb, (((1,), (1,)), ((), ())), preferred_element_type=jnp.float32
    )
    asum = jnp.sum(a, axis=1, keepdims=True)
    vlad = agg - asum * c_ref[...]
    inorm = jnp.sqrt(jnp.sum(vlad * vlad, axis=1, keepdims=True))
    vlad = vlad / jnp.maximum(inorm, EPS)
    g = jnp.sqrt(jnp.sum(vlad * vlad))
    vlad = vlad / jnp.maximum(g, EPS)
    out_ref[0] = vlad


def kernel(x, conv_w, centroids):
    N, D, H, W = x.shape
    K = conv_w.shape[0]
    P = H * W
    xf = x.reshape(N, D, P)
    out = pl.pallas_call(
        _netvlad_body,
        grid=(N,),
        in_specs=[
            pl.BlockSpec((1, D, P), lambda n: (n, 0, 0)),
            pl.BlockSpec((K, D), lambda n: (0, 0)),
            pl.BlockSpec((K, D), lambda n: (0, 0)),
        ],
        out_specs=pl.BlockSpec((1, K, D), lambda n: (n, 0, 0)),
        out_shape=jax.ShapeDtypeStruct((N, K, D), jnp.float32),
        compiler_params=pltpu.CompilerParams(
            dimension_semantics=("parallel",),
        ),
    )(xf, conv_w, centroids)
    return out.reshape(N, K * D)


# speedup vs baseline: 2.2169x; 2.2169x over previous
import functools

import jax
import jax.numpy as jnp
from jax.experimental import pallas as pl
from jax.experimental.pallas import tpu as pltpu

EPS = 1e-12


def _netvlad_body(x_ref, w_ref, c_ref, out_ref, *, P, D, K):
    xp = x_ref[0].reshape(P, D)
    xsq = xp * xp
    ones_row = jnp.ones((1, D), dtype=jnp.float32)
    s2 = jax.lax.dot_general(
        ones_row, xsq, (((1,), (1,)), ((), ())),
        preferred_element_type=jnp.float32,
    )
    rs = 1.0 / jnp.maximum(jnp.sqrt(s2), EPS)
    logits = jax.lax.dot_general(
        w_ref[...], xp, (((1,), (1,)), ((), ())),
        preferred_element_type=jnp.float32,
    )
    logits = logits * rs
    m = jnp.max(logits, axis=0, keepdims=True)
    e = jnp.exp(logits - m)
    a = e * (1.0 / jnp.sum(e, axis=0, keepdims=True))
    asum = jnp.sum(a, axis=1, keepdims=True)
    b = a * rs
    agg = jax.lax.dot_general(
        b, xp, (((1,), (0,)), ((), ())),
        preferred_element_type=jnp.float32,
    )
    vlad = agg - asum * c_ref[...]
    inorm = jnp.sqrt(jnp.sum(vlad * vlad, axis=1, keepdims=True))
    vlad = vlad / jnp.maximum(inorm, EPS)
    g = jnp.sqrt(jnp.sum(vlad * vlad))
    vlad = vlad / jnp.maximum(g, EPS)
    out_ref[0] = vlad


def kernel(x, conv_w, centroids):
    N, D, H, W = x.shape
    K = conv_w.shape[0]
    P = H * W
    xt = jnp.transpose(x, (0, 2, 3, 1))
    out = pl.pallas_call(
        functools.partial(_netvlad_body, P=P, D=D, K=K),
        grid=(N,),
        in_specs=[
            pl.BlockSpec((1, H, W, D), lambda n: (n, 0, 0, 0)),
            pl.BlockSpec((K, D), lambda n: (0, 0)),
            pl.BlockSpec((K, D), lambda n: (0, 0)),
        ],
        out_specs=pl.BlockSpec((1, K, D), lambda n: (n, 0, 0)),
        out_shape=jax.ShapeDtypeStruct((N, K, D), jnp.float32),
        compiler_params=pltpu.CompilerParams(
            dimension_semantics=("arbitrary",),
        ),
    )(xt, conv_w, centroids)
    return out.reshape(N, K * D)


# speedup vs baseline: 3.2907x; 1.4843x over previous
import functools

import jax
import jax.numpy as jnp
from jax.experimental import pallas as pl
from jax.experimental.pallas import tpu as pltpu

EPS = 1e-12


def _netvlad_body(x_ref, w_ref, c_ref, out_ref, *, P, D, K, NB):
    for i in range(NB):
        xp = x_ref[i].reshape(P, D)
        xsq = xp * xp
        ones_row = jnp.ones((1, D), dtype=jnp.float32)
        s2 = jax.lax.dot_general(
            ones_row, xsq, (((1,), (1,)), ((), ())),
            preferred_element_type=jnp.float32,
        )
        rs = 1.0 / jnp.maximum(jnp.sqrt(s2), EPS)
        logits = jax.lax.dot_general(
            w_ref[...], xp, (((1,), (1,)), ((), ())),
            preferred_element_type=jnp.float32,
        )
        logits = logits * rs
        m = jnp.max(logits, axis=0, keepdims=True)
        e = jnp.exp(logits - m)
        a = e * (1.0 / jnp.sum(e, axis=0, keepdims=True))
        asum = jnp.sum(a, axis=1, keepdims=True)
        b = a * rs
        agg = jax.lax.dot_general(
            b, xp, (((1,), (0,)), ((), ())),
            preferred_element_type=jnp.float32,
        )
        vlad = agg - asum * c_ref[...]
        inorm = jnp.sqrt(jnp.sum(vlad * vlad, axis=1, keepdims=True))
        vlad = vlad / jnp.maximum(inorm, EPS)
        g = jnp.sqrt(jnp.sum(vlad * vlad))
        vlad = vlad / jnp.maximum(g, EPS)
        out_ref[i] = vlad


def kernel(x, conv_w, centroids):
    N, D, H, W = x.shape
    K = conv_w.shape[0]
    P = H * W
    NB = 8
    xt = jnp.transpose(x, (0, 2, 3, 1))
    out = pl.pallas_call(
        functools.partial(_netvlad_body, P=P, D=D, K=K, NB=NB),
        grid=(N // NB,),
        in_specs=[
            pl.BlockSpec((NB, H, W, D), lambda n: (n, 0, 0, 0)),
            pl.BlockSpec((K, D), lambda n: (0, 0)),
            pl.BlockSpec((K, D), lambda n: (0, 0)),
        ],
        out_specs=pl.BlockSpec((NB, K, D), lambda n: (n, 0, 0)),
        out_shape=jax.ShapeDtypeStruct((N, K, D), jnp.float32),
        compiler_params=pltpu.CompilerParams(
            dimension_semantics=("arbitrary",),
            vmem_limit_bytes=56 * 1024 * 1024,
        ),
    )(xt, conv_w, centroids)
    return out.reshape(N, K * D)
